# baseline (speedup 1.0000x reference)
import functools

import jax
import jax.numpy as jnp
from jax import lax
from jax.experimental import pallas as pl
from jax.experimental.pallas import tpu as pltpu
from jax.experimental.pallas import tpu_sc as plsc

_N_COARSE = 50000
_N_FINE = 100000
_D = 128
_K = 3
_CH = 128
_NW = 32
_NFULL = _N_FINE // _CH
_TAIL = _N_FINE - _NFULL * _CH
_NPAIRS = 12
_L = 16

_mesh = plsc.VectorSubcoreMesh(core_axis_name="c", subcore_axis_name="s")


def _write_positions(pos_ref, base, n):
    lanes = lax.iota(jnp.int32, _L)
    for j in range(n // _L):
        pos_ref[pl.ds(j * _L, _L)] = (lanes + (base + j * _L)) * _K


def _clamp_detect(idx_ref, save_ref, n):
    clamp = jnp.full((_L,), _N_COARSE - 1, jnp.int32)
    one = jnp.ones((_L,), jnp.int32)
    zero = jnp.zeros((_L,), jnp.int32)
    shadow = zero
    for j in range(n // _L):
        v = idx_ref[pl.ds(j * _L, _L)]
        save_ref[pl.ds(j * _L, _L)] = v
        shadow = shadow + jnp.where(v >= _N_COARSE, one, zero)
        idx_ref[pl.ds(j * _L, _L)] = jnp.minimum(v, clamp)
    total = shadow[0]
    for l in range(1, _L):
        total = total + shadow[l]
    return total > 0


def _zero_shadow_rows(save_ref, rows_ref, n):
    zero = jnp.zeros((_L,), jnp.float32)

    def body(i, carry):
        orig = save_ref[pl.ds(i, _L)][0]

        @pl.when(orig == _N_COARSE)
        def _():
            for k in range(_D // _L):
                rows_ref[i, pl.ds(k * _L, _L)] = zero
        return carry

    lax.fori_loop(0, n, body, 0)


@functools.partial(
    pl.kernel,
    out_type=jax.ShapeDtypeStruct((_N_FINE, _D), jnp.float32),
    mesh=_mesh,
    scratch_types=[
        pltpu.VMEM((2, _CH), jnp.int32),
        pltpu.VMEM((2, _CH), jnp.int32),
        pltpu.VMEM((2, _CH, _D), jnp.float32),
        pltpu.VMEM((_CH + _L,), jnp.int32),
        pltpu.VMEM((_TAIL,), jnp.int32),
        pltpu.VMEM((_TAIL,), jnp.int32),
        pltpu.VMEM((_TAIL, _D), jnp.float32),
        pltpu.SemaphoreType.DMA,
        pltpu.SemaphoreType.DMA,
        pltpu.SemaphoreType.DMA,
        pltpu.SemaphoreType.DMA,
        pltpu.SemaphoreType.DMA,
        pltpu.SemaphoreType.DMA,
        pltpu.SemaphoreType.DMA,
    ],
)
def _sc_gather(x_hbm, ups_hbm, out_hbm, pos_v, idx_v, rows_v, idx_s,
               pos_t, idx_t, rows_t, si0, si1, sg0, sg1, sw0, sw1, st):
    wid = lax.axis_index("s") * 2 + lax.axis_index("c")
    nc = jnp.where(wid <= 12, 25, 24)

    sem_i = (si0, si1)
    sem_g = (sg0, sg1)
    sem_w = (sw0, sw1)

    def chunk_step(c, b):
        chunk = wid + c * _NW
        base = chunk * _CH
        my_pos = pos_v.at[b]
        my_idx = idx_v.at[b]
        my_rows = rows_v.at[b]

        @pl.when(c == 0)
        def _():
            _write_positions(my_pos, base, _CH)
            pltpu.async_copy(ups_hbm.at[my_pos], my_idx, sem_i[b])
            _write_positions(pos_v.at[1 - b], base + _NW * _CH, _CH)
            pltpu.async_copy(ups_hbm.at[pos_v.at[1 - b]], idx_v.at[1 - b],
                             sem_i[1 - b])

        pltpu.make_async_copy(ups_hbm.at[my_pos], my_idx, sem_i[b]).wait()

        bad = _clamp_detect(my_idx, idx_s, _CH)

        @pl.when(c >= 2)
        def _():
            pltpu.make_async_copy(my_rows, out_hbm.at[pl.ds(0, _CH)],
                                  sem_w[b]).wait()

        gather = pltpu.async_copy(x_hbm.at[my_idx], my_rows, sem_g[b])
        gather.wait()

        @pl.when(bad)
        def _():
            _zero_shadow_rows(idx_s, my_rows, _CH)

        @pl.when(c + 2 < nc)
        def _():
            _write_positions(my_pos, base + 2 * _NW * _CH, _CH)
            pltpu.async_copy(ups_hbm.at[my_pos], my_idx, sem_i[b])

        pltpu.async_copy(my_rows, out_hbm.at[pl.ds(base, _CH)], sem_w[b])

    def pair_body(p, carry):
        chunk_step(2 * p, 0)
        chunk_step(2 * p + 1, 1)
        return carry

    lax.fori_loop(0, _NPAIRS, pair_body, 0)

    @pl.when(nc == 25)
    def _():
        chunk_step(jnp.int32(24), 0)

    pltpu.make_async_copy(rows_v.at[0], out_hbm.at[pl.ds(0, _CH)], sw0).wait()
    pltpu.make_async_copy(rows_v.at[1], out_hbm.at[pl.ds(0, _CH)], sw1).wait()

    @pl.when(wid == _NW - 1)
    def _():
        tbase = _NFULL * _CH
        _write_positions(pos_t, tbase, _TAIL)
        pltpu.async_copy(ups_hbm.at[pos_t], idx_t, st).wait()
        tbad = _clamp_detect(idx_t, idx_s, _TAIL)
        pltpu.async_copy(x_hbm.at[idx_t], rows_t, st).wait()

        @pl.when(tbad)
        def _():
            _zero_shadow_rows(idx_s, rows_t, _TAIL)

        pltpu.sync_copy(rows_t, out_hbm.at[pl.ds(tbase, _TAIL)])


def kernel(x, upsamples):
    return _sc_gather(x, upsamples.reshape(-1))


# speedup vs baseline: 2.3436x; 2.3436x over previous
import functools

import jax
import jax.numpy as jnp
from jax import lax
from jax.experimental import pallas as pl
from jax.experimental.pallas import tpu as pltpu
from jax.experimental.pallas import tpu_sc as plsc

_N_COARSE = 50000
_N_FINE = 100000
_D = 128
_CH = 128
_NW = 32
_NFULL = _N_FINE // _CH
_TAIL = _N_FINE - _NFULL * _CH
_NTRIPLES = 8
_L = 16

_mesh = plsc.VectorSubcoreMesh(core_axis_name="c", subcore_axis_name="s")


def _clamp_detect(idx_ref, save_ref, n):
    clamp = jnp.full((_L,), _N_COARSE - 1, jnp.int32)
    one = jnp.ones((_L,), jnp.int32)
    zero = jnp.zeros((_L,), jnp.int32)
    shadow = zero
    for j in range(n // _L):
        v = idx_ref[pl.ds(j * _L, _L)]
        save_ref[pl.ds(j * _L, _L)] = v
        shadow = shadow + jnp.where(v >= _N_COARSE, one, zero)
        idx_ref[pl.ds(j * _L, _L)] = jnp.minimum(v, clamp)
    total = shadow[0]
    for l in range(1, _L):
        total = total + shadow[l]
    return jnp.where(total > 0, jnp.int32(1), jnp.int32(0))


def _zero_shadow_rows(save_ref, rows_ref, n):
    zero = jnp.zeros((_L,), jnp.float32)

    def body(i, carry):
        orig = save_ref[pl.ds(i, _L)][0]

        @pl.when(orig == _N_COARSE)
        def _():
            for k in range(_D // _L):
                rows_ref[i, pl.ds(k * _L, _L)] = zero
        return carry

    lax.fori_loop(0, n, body, 0)


@functools.partial(
    pl.kernel,
    out_type=jax.ShapeDtypeStruct((_N_FINE, _D), jnp.float32),
    mesh=_mesh,
    scratch_types=[
        pltpu.VMEM((3, _CH), jnp.int32),
        pltpu.VMEM((3, _CH, _D), jnp.float32),
        pltpu.VMEM((_CH + _L,), jnp.int32),
        pltpu.VMEM((_CH + _L,), jnp.int32),
        pltpu.VMEM((_CH + _L,), jnp.int32),
        pltpu.VMEM((_TAIL,), jnp.int32),
        pltpu.VMEM((_TAIL + _L,), jnp.int32),
        pltpu.VMEM((_TAIL, _D), jnp.float32),
        pltpu.SMEM((4,), jnp.int32),
        pltpu.SemaphoreType.DMA,
        pltpu.SemaphoreType.DMA,
        pltpu.SemaphoreType.DMA,
        pltpu.SemaphoreType.DMA,
        pltpu.SemaphoreType.DMA,
        pltpu.SemaphoreType.DMA,
        pltpu.SemaphoreType.DMA,
        pltpu.SemaphoreType.DMA,
        pltpu.SemaphoreType.DMA,
        pltpu.SemaphoreType.DMA,
    ],
)
def _sc_gather(x_hbm, idx_hbm, out_hbm, idx_v, rows_v, idx_s0, idx_s1, idx_s2,
               idx_t, save_t, rows_t, flags,
               si0, si1, si2, sg0, sg1, sg2, sw0, sw1, sw2, st):
    idx_s = (idx_s0, idx_s1, idx_s2)
    wid = lax.axis_index("s") * 2 + lax.axis_index("c")
    nc = jnp.where(wid <= 12, 25, 24)

    sem_i = (si0, si1, si2)
    sem_g = (sg0, sg1, sg2)
    sem_w = (sw0, sw1, sw2)

    def finish_chunk(c, b):
        base = (wid + c * _NW) * _CH
        my_rows = rows_v.at[b]
        pltpu.make_async_copy(x_hbm.at[idx_v.at[b]], my_rows,
                              sem_g[b]).wait()

        @pl.when(flags[b] != 0)
        def _():
            _zero_shadow_rows(idx_s[b], my_rows, _CH)

        pltpu.async_copy(my_rows, out_hbm.at[pl.ds(base, _CH)], sem_w[b])

        @pl.when(c + 3 < nc)
        def _():
            pltpu.async_copy(
                idx_hbm.at[pl.ds(base + 3 * _NW * _CH, _CH)], idx_v.at[b],
                sem_i[b])

    def chunk_step(c, b):
        chunk = wid + c * _NW
        base = chunk * _CH
        my_idx = idx_v.at[b]
        my_rows = rows_v.at[b]

        @pl.when(c == 0)
        def _():
            for s in range(3):
                pltpu.async_copy(
                    idx_hbm.at[pl.ds(base + s * _NW * _CH, _CH)],
                    idx_v.at[s], sem_i[s])

        pltpu.make_async_copy(idx_hbm.at[pl.ds(0, _CH)], my_idx,
                              sem_i[b]).wait()

        flags[b] = _clamp_detect(my_idx, idx_s[b], _CH)

        @pl.when(c >= 3)
        def _():
            pltpu.make_async_copy(my_rows, out_hbm.at[pl.ds(0, _CH)],
                                  sem_w[b]).wait()

        pltpu.async_copy(x_hbm.at[my_idx], my_rows, sem_g[b])

        @pl.when(c >= 1)
        def _():
            finish_chunk(c - 1, (b + 2) % 3)

    def triple_body(p, carry):
        chunk_step(3 * p, 0)
        chunk_step(3 * p + 1, 1)
        chunk_step(3 * p + 2, 2)
        return carry

    lax.fori_loop(0, _NTRIPLES, triple_body, 0)

    @pl.when(nc == 25)
    def _():
        chunk_step(jnp.int32(24), 0)

    @pl.when(nc == 25)
    def _():
        finish_chunk(jnp.int32(24), 0)

    @pl.when(nc == 24)
    def _():
        finish_chunk(jnp.int32(23), 2)

    for s, sw in enumerate((sw0, sw1, sw2)):
        pltpu.make_async_copy(rows_v.at[s], out_hbm.at[pl.ds(0, _CH)],
                              sw).wait()

    @pl.when(wid == _NW - 1)
    def _():
        tbase = _NFULL * _CH
        pltpu.sync_copy(idx_hbm.at[pl.ds(tbase, _TAIL)], idx_t)
        tbad = _clamp_detect(idx_t, save_t, _TAIL)
        pltpu.async_copy(x_hbm.at[idx_t], rows_t, st).wait()

        @pl.when(tbad != 0)
        def _():
            _zero_shadow_rows(save_t, rows_t, _TAIL)

        pltpu.sync_copy(rows_t, out_hbm.at[pl.ds(tbase, _TAIL)])


def kernel(x, upsamples):
    return _sc_gather(x, upsamples[:, 0])
